# baseline (speedup 1.0000x reference)
import functools

import jax
import jax.numpy as jnp
from jax import lax
from jax.experimental import pallas as pl
from jax.experimental.pallas import tpu as pltpu
from jax.experimental.pallas import tpu_sc as plsc

_K = 8192
_C = 32
_B = 4
_HW = 1024
_N = _B * _HW
_TK = 2048
_BETA = 0.25


def _argmin_body(z3m_ref, w_ref, szt_ref, idx_ref, minsum_ref, sw_ref):
    sub_iota = lax.broadcasted_iota(jnp.int32, (8, _HW), 0).astype(jnp.float32)
    nstrip = _TK // 8
    total = jnp.zeros((1, 1), jnp.float32)
    wf = w_ref[...]
    sw_ref[...] = jnp.sum(wf * wf, axis=1, keepdims=True)

    for b in range(_B):
        zb = z3m_ref[b]
        szb = szt_ref[b]

        def body(j, carry, zb=zb, szb=szb):
            run_min, run_idx = carry
            wt = w_ref[pl.ds(j * _TK, _TK), :]
            mm = lax.dot_general(wt, zb, (((1,), (0,)), ((), ())),
                                 preferred_element_type=jnp.float32)
            d = (szb + mm) + sw_ref[pl.ds(j * _TK, _TK), :]
            acc_v = d[0:8]
            acc_i = jnp.zeros((8, _HW), jnp.float32)
            for s in range(1, nstrip):
                strip = d[8 * s:8 * (s + 1)]
                upd = strip < acc_v
                acc_v = jnp.where(upd, strip, acc_v)
                acc_i = jnp.where(upd, jnp.float32(s), acc_i)
            tmin = jnp.min(acc_v, axis=0, keepdims=True)
            grow = acc_i * 8.0 + sub_iota
            targf = jnp.min(jnp.where(acc_v == tmin, grow, jnp.float32(1e9)),
                            axis=0, keepdims=True)
            targ = targf.astype(jnp.int32) + j * _TK
            upd = tmin < run_min
            return (jnp.where(upd, tmin, run_min),
                    jnp.where(upd, targ, run_idx))

        init = (jnp.full((1, _HW), jnp.inf, jnp.float32),
                jnp.zeros((1, _HW), jnp.int32))
        run_min, run_idx = lax.fori_loop(0, _K // _TK, body, init)
        idx_ref[:, pl.ds(b * _HW, _HW)] = run_idx
        total = total + jnp.sum(run_min, keepdims=True).reshape(1, 1)

    minsum_ref[...] = total


_NC = 2
_NS = 16
_NW = _NC * _NS
_BPW = _N // _NW


@functools.cache
def _make_sc_gather():
    mesh = plsc.VectorSubcoreMesh(core_axis_name="c", subcore_axis_name="s")

    @functools.partial(
        pl.kernel,
        mesh=mesh,
        out_type=jax.ShapeDtypeStruct((_N, _C), jnp.float32),
        scratch_types=[
            pltpu.VMEM((_BPW,), jnp.int32),
            pltpu.VMEM((_BPW, _C), jnp.float32),
            pltpu.SemaphoreType.DMA,
        ],
        compiler_params=pltpu.CompilerParams(use_tc_tiling_on_sc=False),
    )
    def _sc_gather(table_hbm, idx_hbm, out_hbm, idx_v, rows_v, sem):
        wid = lax.axis_index("s") * _NC + lax.axis_index("c")
        base = wid * _BPW
        pltpu.sync_copy(idx_hbm.at[pl.ds(base, _BPW)], idx_v)
        pltpu.async_copy(table_hbm.at[idx_v], rows_v, sem).wait()
        pltpu.sync_copy(rows_v, out_hbm.at[pl.ds(base, _BPW)])

    return _sc_gather


def _assemble_body(q_ref, out_ref):
    eye = (lax.broadcasted_iota(jnp.int32, (_C, _C), 0)
           == lax.broadcasted_iota(jnp.int32, (_C, _C), 1)).astype(jnp.float32)
    for b in range(_B):
        qb = q_ref[pl.ds(b * _HW, _HW), :]
        out_ref[b] = lax.dot_general(eye, qb, (((1,), (1,)), ((), ())),
                                     preferred_element_type=jnp.float32)


def kernel(z, W):
    z3 = z.reshape(_B, _C, _HW)
    z3m = -2.0 * z3
    fz = jnp.transpose(z, (0, 2, 3, 1)).reshape(-1, _C)
    szt = jnp.sum(fz ** 2, axis=1).reshape(_B, 1, _HW)

    idx2, minsum = pl.pallas_call(
        _argmin_body,
        out_shape=(jax.ShapeDtypeStruct((1, _N), jnp.int32),
                   jax.ShapeDtypeStruct((1, 1), jnp.float32)),
        scratch_shapes=[pltpu.VMEM((_K, 1), jnp.float32)],
    )(z3m, W, szt)

    q = _make_sc_gather()(W, idx2.reshape(_N))

    out3 = pl.pallas_call(
        _assemble_body,
        out_shape=jax.ShapeDtypeStruct((_B, _C, _HW), jnp.float32),
    )(q)
    out = out3.reshape(_B, _C, 32, 32)
    loss = (1.0 + _BETA) * (minsum[0, 0] / (_N * _C))
    return (out, loss)


# speedup vs baseline: 1.0964x; 1.0964x over previous
import functools

import jax
import jax.numpy as jnp
from jax import lax
from jax.experimental import pallas as pl
from jax.experimental.pallas import tpu as pltpu
from jax.experimental.pallas import tpu_sc as plsc

_K = 8192
_C = 32
_B = 4
_HW = 1024
_N = _B * _HW
_TK = 4096
_BETA = 0.25


def _argmin_body(z3m_ref, w_ref, szt_ref, idx_ref, minsum_ref, sw_ref):
    sub_iota = lax.broadcasted_iota(jnp.int32, (8, _HW), 0).astype(jnp.float32)
    nstrip = _TK // 8
    total = jnp.zeros((1, 1), jnp.float32)
    wf = w_ref[...]
    sw_ref[...] = jnp.sum(wf * wf, axis=1, keepdims=True)

    for b in range(_B):
        zb = z3m_ref[b]
        szb = szt_ref[b]

        def body(j, carry, zb=zb, szb=szb):
            run_min, run_idx = carry
            wt = w_ref[pl.ds(j * _TK, _TK), :]
            mm = lax.dot_general(wt, zb, (((1,), (0,)), ((), ())),
                                 preferred_element_type=jnp.float32)
            d = (szb + mm) + sw_ref[pl.ds(j * _TK, _TK), :]
            acc_v = d[0:8]
            acc_i = jnp.zeros((8, _HW), jnp.float32)
            for s in range(1, nstrip):
                strip = d[8 * s:8 * (s + 1)]
                upd = strip < acc_v
                acc_v = jnp.where(upd, strip, acc_v)
                acc_i = jnp.where(upd, jnp.float32(s), acc_i)
            tmin = jnp.min(acc_v, axis=0, keepdims=True)
            grow = acc_i * 8.0 + sub_iota
            targf = jnp.min(jnp.where(acc_v == tmin, grow, jnp.float32(1e9)),
                            axis=0, keepdims=True)
            targ = targf.astype(jnp.int32) + j * _TK
            upd = tmin < run_min
            return (jnp.where(upd, tmin, run_min),
                    jnp.where(upd, targ, run_idx))

        init = (jnp.full((1, _HW), jnp.inf, jnp.float32),
                jnp.zeros((1, _HW), jnp.int32))
        run_min, run_idx = lax.fori_loop(0, _K // _TK, body, init)
        idx_ref[:, pl.ds(b * _HW, _HW)] = run_idx
        total = total + jnp.sum(run_min, keepdims=True).reshape(1, 1)

    minsum_ref[...] = total


_NC = 2
_NS = 16
_NW = _NC * _NS
_BPW = _N // _NW


@functools.cache
def _make_sc_gather():
    mesh = plsc.VectorSubcoreMesh(core_axis_name="c", subcore_axis_name="s")

    @functools.partial(
        pl.kernel,
        mesh=mesh,
        out_type=jax.ShapeDtypeStruct((_N, _C), jnp.float32),
        scratch_types=[
            pltpu.VMEM((_BPW,), jnp.int32),
            pltpu.VMEM((_BPW, _C), jnp.float32),
            pltpu.SemaphoreType.DMA,
        ],
        compiler_params=pltpu.CompilerParams(use_tc_tiling_on_sc=False),
    )
    def _sc_gather(table_hbm, idx_hbm, out_hbm, idx_v, rows_v, sem):
        wid = lax.axis_index("s") * _NC + lax.axis_index("c")
        base = wid * _BPW
        pltpu.sync_copy(idx_hbm.at[pl.ds(base, _BPW)], idx_v)
        pltpu.async_copy(table_hbm.at[idx_v], rows_v, sem).wait()
        pltpu.sync_copy(rows_v, out_hbm.at[pl.ds(base, _BPW)])

    return _sc_gather


def _assemble_body(q_ref, out_ref):
    eye = (lax.broadcasted_iota(jnp.int32, (_C, _C), 0)
           == lax.broadcasted_iota(jnp.int32, (_C, _C), 1)).astype(jnp.float32)
    for b in range(_B):
        qb = q_ref[pl.ds(b * _HW, _HW), :]
        out_ref[b] = lax.dot_general(eye, qb, (((1,), (1,)), ((), ())),
                                     preferred_element_type=jnp.float32)


def kernel(z, W):
    z3 = z.reshape(_B, _C, _HW)
    z3m = -2.0 * z3
    fz = jnp.transpose(z, (0, 2, 3, 1)).reshape(-1, _C)
    szt = jnp.sum(fz ** 2, axis=1).reshape(_B, 1, _HW)

    idx2, minsum = pl.pallas_call(
        _argmin_body,
        out_shape=(jax.ShapeDtypeStruct((1, _N), jnp.int32),
                   jax.ShapeDtypeStruct((1, 1), jnp.float32)),
        scratch_shapes=[pltpu.VMEM((_K, 1), jnp.float32)],
    )(z3m, W, szt)

    q = _make_sc_gather()(W, idx2.reshape(_N))

    out = jnp.transpose(q.reshape(_B, 32, 32, _C), (0, 3, 1, 2))
    loss = (1.0 + _BETA) * (minsum[0, 0] / (_N * _C))
    return (out, loss)


# speedup vs baseline: 1.1141x; 1.0161x over previous
import functools

import jax
import jax.numpy as jnp
from jax import lax
from jax.experimental import pallas as pl
from jax.experimental.pallas import tpu as pltpu
from jax.experimental.pallas import tpu_sc as plsc

_K = 8192
_C = 32
_B = 4
_HW = 1024
_N = _B * _HW
_TK = 8192
_BETA = 0.25


def _argmin_body(z3m_ref, w_ref, szt_ref, idx_ref, minsum_ref, sw_ref):
    sub_iota = lax.broadcasted_iota(jnp.int32, (8, _HW), 0).astype(jnp.float32)
    nstrip = _TK // 8
    total = jnp.zeros((1, 1), jnp.float32)
    wf = w_ref[...]
    sw_ref[...] = jnp.sum(wf * wf, axis=1, keepdims=True)

    for b in range(_B):
        zb = z3m_ref[b]
        szb = szt_ref[b]

        def body(j, carry, zb=zb, szb=szb):
            run_min, run_idx = carry
            wt = w_ref[pl.ds(j * _TK, _TK), :]
            mm = lax.dot_general(wt, zb, (((1,), (0,)), ((), ())),
                                 preferred_element_type=jnp.float32)
            d = (szb + mm) + sw_ref[pl.ds(j * _TK, _TK), :]
            acc_v = d[0:8]
            acc_i = jnp.zeros((8, _HW), jnp.float32)
            for s in range(1, nstrip):
                strip = d[8 * s:8 * (s + 1)]
                upd = strip < acc_v
                acc_v = jnp.where(upd, strip, acc_v)
                acc_i = jnp.where(upd, jnp.float32(s), acc_i)
            tmin = jnp.min(acc_v, axis=0, keepdims=True)
            grow = acc_i * 8.0 + sub_iota
            targf = jnp.min(jnp.where(acc_v == tmin, grow, jnp.float32(1e9)),
                            axis=0, keepdims=True)
            targ = targf.astype(jnp.int32) + j * _TK
            upd = tmin < run_min
            return (jnp.where(upd, tmin, run_min),
                    jnp.where(upd, targ, run_idx))

        init = (jnp.full((1, _HW), jnp.inf, jnp.float32),
                jnp.zeros((1, _HW), jnp.int32))
        run_min, run_idx = lax.fori_loop(0, _K // _TK, body, init)
        idx_ref[:, pl.ds(b * _HW, _HW)] = run_idx
        total = total + jnp.sum(run_min, keepdims=True).reshape(1, 1)

    minsum_ref[...] = total


_NC = 2
_NS = 16
_NW = _NC * _NS
_BPW = _N // _NW


@functools.cache
def _make_sc_gather():
    mesh = plsc.VectorSubcoreMesh(core_axis_name="c", subcore_axis_name="s")

    @functools.partial(
        pl.kernel,
        mesh=mesh,
        out_type=jax.ShapeDtypeStruct((_N, _C), jnp.float32),
        scratch_types=[
            pltpu.VMEM((_BPW,), jnp.int32),
            pltpu.VMEM((_BPW, _C), jnp.float32),
            pltpu.SemaphoreType.DMA,
        ],
        compiler_params=pltpu.CompilerParams(use_tc_tiling_on_sc=False),
    )
    def _sc_gather(table_hbm, idx_hbm, out_hbm, idx_v, rows_v, sem):
        wid = lax.axis_index("s") * _NC + lax.axis_index("c")
        base = wid * _BPW
        pltpu.sync_copy(idx_hbm.at[pl.ds(base, _BPW)], idx_v)
        pltpu.async_copy(table_hbm.at[idx_v], rows_v, sem).wait()
        pltpu.sync_copy(rows_v, out_hbm.at[pl.ds(base, _BPW)])

    return _sc_gather


def _assemble_body(q_ref, out_ref):
    eye = (lax.broadcasted_iota(jnp.int32, (_C, _C), 0)
           == lax.broadcasted_iota(jnp.int32, (_C, _C), 1)).astype(jnp.float32)
    for b in range(_B):
        qb = q_ref[pl.ds(b * _HW, _HW), :]
        out_ref[b] = lax.dot_general(eye, qb, (((1,), (1,)), ((), ())),
                                     preferred_element_type=jnp.float32)


def kernel(z, W):
    z3 = z.reshape(_B, _C, _HW)
    z3m = -2.0 * z3
    fz = jnp.transpose(z, (0, 2, 3, 1)).reshape(-1, _C)
    szt = jnp.sum(fz ** 2, axis=1).reshape(_B, 1, _HW)

    idx2, minsum = pl.pallas_call(
        _argmin_body,
        out_shape=(jax.ShapeDtypeStruct((1, _N), jnp.int32),
                   jax.ShapeDtypeStruct((1, 1), jnp.float32)),
        scratch_shapes=[pltpu.VMEM((_K, 1), jnp.float32)],
    )(z3m, W, szt)

    q = _make_sc_gather()(W, idx2.reshape(_N))

    out = jnp.transpose(q.reshape(_B, 32, 32, _C), (0, 3, 1, 2))
    loss = (1.0 + _BETA) * (minsum[0, 0] / (_N * _C))
    return (out, loss)
